# baseline (speedup 1.0000x reference)
import numpy as np
import jax
import jax.numpy as jnp
from jax.experimental import pallas as pl
from jax.experimental.pallas import tpu as pltpu

_N = 100
_S = 6 * _N + 12
_NTRANS = 3 * _N + 5
_NEMIT = 126
_EMITC = 6 ** 3

_HIGH = jax.lax.Precision.HIGHEST


def _static_tables():
    n, S = _N, _S
    const = np.zeros((S, S), np.float32)
    mask = np.zeros((S, S), np.float32)
    ma = np.zeros((S, S), np.float32)
    mb = np.zeros((S, S), np.float32)
    mc = np.zeros((S, S), np.float32)
    mn = np.zeros((S, S), np.float32)
    mdel = np.zeros((S, S), np.float32)
    ke = np.ones((S, S), np.float32)
    selA = np.zeros((S, _NTRANS), np.float32)
    selB = np.zeros((S, _NTRANS), np.float32)
    selC = np.zeros((S, _NTRANS), np.float32)
    selN = np.zeros((S, _NTRANS), np.float32)

    def ent(r, c):
        mask[r, c] = 1.0

    ent(0, 0); const[0, 0] = 1.0; mn[0, 0] = 1.0; selN[0, 0] = 1.0
    ent(0, 1); ma[0, 1] = 1.0; selA[0, 0] = 1.0
    ent(1, 2); const[1, 2] = 1.0
    ent(2, 3); const[2, 3] = 1.0
    for i in range(n):
        ent(3 + 3 * i, 4 + 3 * i)
        ma[3 + 3 * i, 4 + 3 * i] = 1.0
        selA[3 + 3 * i, 1 + i] = 1.0
        ent(4 + 3 * i, 5 + 3 * i); const[4 + 3 * i, 5 + 3 * i] = 1.0
        ent(5 + 3 * i, 6 + 3 * i); const[5 + 3 * i, 6 + 3 * i] = 1.0
    off = 8 + 3 * n
    for i in range(n + 1):
        ent(3 + 3 * i, off + 3 * i)
        mb[3 + 3 * i, off + 3 * i] = 1.0
        selB[3 + 3 * i, 101 + i] = 1.0
        ent(off + 3 * i, off + 1 + 3 * i); const[off + 3 * i, off + 1 + 3 * i] = 1.0
        ent(off + 1 + 3 * i, off + 2 + 3 * i); const[off + 1 + 3 * i, off + 2 + 3 * i] = 1.0
        ent(off + 2 + 3 * i, 4 + 3 * i)
        mc[off + 2 + 3 * i, 4 + 3 * i] = 1.0
        selC[off + 2 + 3 * i, 203 + i] = 1.0
        ent(off + 2 + 3 * i, off + 3 * i)
        const[off + 2 + 3 * i, off + 3 * i] = 1.0
        mn[off + 2 + 3 * i, off + 3 * i] = 1.0
        selN[off + 2 + 3 * i, 203 + i] = 1.0
    ent(303, 304); ma[303, 304] = 1.0; selA[303, 202] = 1.0
    for i in range(n):
        for j in range(i + 1, n + 1):
            r, c = 3 + 3 * i, 4 + 3 * j
            ent(r, c)
            const[r, c] = 1.0
            mdel[r, c] = 1.0
            ke[r, c] = float(1 + (j - i))
    t1 = 8 + 3 * n + 3 * (n + 1)
    for r, c in ((304, 305), (305, 306), (306, 307), (307, 307), (307, t1), (t1, t1)):
        ent(r, c); const[r, c] = 1.0
    return const, mask, ma, mb, mc, mn, mdel, ke, selA, selB, selC, selN


_TABLES = _static_tables()


def _cell_body(inp_ref, alpha_ref, count_ref, loglik_ref, w_ref, ek_ref, ik_ref,
               const_ref, mask_ref, ma_ref, mb_ref, mc_ref, mn_ref, mdel_ref,
               ke_ref, selA_ref, selB_ref, selC_ref, selN_ref,
               alpha_out_ref, count_out_ref, loglik_out_ref):
    w = w_ref[...]
    gA = jnp.dot(selA_ref[...], w, preferred_element_type=jnp.float32, precision=_HIGH)
    gB = jnp.dot(selB_ref[...], w, preferred_element_type=jnp.float32, precision=_HIGH)
    gC = jnp.dot(selC_ref[...], w, preferred_element_type=jnp.float32, precision=_HIGH)
    gN = jnp.dot(selN_ref[...], w, preferred_element_type=jnp.float32, precision=_HIGH)

    w304 = w_ref[304, 0]
    loga = jnp.log(jnp.abs(w304))
    sgn = jnp.sign(w304)
    ke = ke_ref[...]
    odd = ke - 2.0 * jnp.floor(ke * 0.5)
    pw = jnp.exp(ke * loga) * (odd * sgn + (1.0 - odd))

    V = (const_ref[...]
         + ma_ref[...] * gA + mb_ref[...] * gB + mc_ref[...] * gC
         - mn_ref[...] * gN - mdel_ref[...] * pw)

    mask = mask_ref[...]
    gmax = jnp.max(jnp.where(mask > 0.0, V, -jnp.inf))
    E = mask * jnp.exp(V - gmax)
    rowsum = jnp.sum(E, axis=1, keepdims=True)
    A = E * (1.0 / rowsum)

    x = ek_ref[...][:, :_NEMIT]
    xm = jnp.max(x, axis=1, keepdims=True)
    Bexp = jnp.exp(x - xm)
    B = Bexp * (1.0 / jnp.sum(Bexp, axis=1, keepdims=True))
    emis = jax.lax.dot_general(inp_ref[...], B, (((1,), (1,)), ((), ())),
                               preferred_element_type=jnp.float32, precision=_HIGH)

    ik = ik_ref[...]
    ikm = jnp.max(ik)
    pexp = jnp.exp(ik - ikm)
    pi = pexp * (1.0 / jnp.sum(pexp))

    alphaA = jnp.dot(alpha_ref[...], A, preferred_element_type=jnp.float32, precision=_HIGH)
    count = count_ref[...]
    first = count == 0.0
    alpha_new = jnp.where(first, pi, alphaA) * emis
    Z = jnp.sum(alpha_new, axis=1, keepdims=True) + 1e-30
    alpha_out_ref[...] = alpha_new / Z
    count_out_ref[...] = count + 1.0
    loglik_out_ref[...] = loglik_ref[...] + jnp.log(Z)


def kernel(inputs, alpha, count, loglik, transition_kernel, emission_kernel, init_kernel):
    batch = inputs.shape[0]
    w = transition_kernel.reshape(_NTRANS, 1)
    ek = emission_kernel.reshape(_S, _EMITC)
    ik = init_kernel.reshape(1, _S)
    out = pl.pallas_call(
        _cell_body,
        out_shape=(
            jax.ShapeDtypeStruct((batch, _S), jnp.float32),
            jax.ShapeDtypeStruct((batch, 1), jnp.float32),
            jax.ShapeDtypeStruct((batch, 1), jnp.float32),
        ),
    )(inputs, alpha, count, loglik, w, ek, ik,
      *(jnp.asarray(t) for t in _TABLES))
    return out


# speedup vs baseline: 1.1157x; 1.1157x over previous
import numpy as np
import jax
import jax.numpy as jnp
from jax.experimental import pallas as pl
from jax.experimental.pallas import tpu as pltpu

_N = 100
_S = 6 * _N + 12
_NTRANS = 3 * _N + 5
_NTERM = 304
_NEMIT = 126
_EMITC = 6 ** 3

_HIGH = jax.lax.Precision.HIGHEST

_F_CONST = 1
_F_MASK = 2
_F_DEL = 4


def _static_tables():
    n, S = _N, _S
    flags = np.zeros((S, S), np.int8)
    ke = np.ones((S, S), np.int8)
    rowone = np.zeros((S, _NTERM), np.float32)
    colsign = np.zeros((_NTERM, S), np.int8)

    def ent(r, c, const=False):
        flags[r, c] |= _F_MASK
        if const:
            flags[r, c] |= _F_CONST

    def term(t, r, c, sign):
        rowone[r, t] = 1.0
        colsign[t, c] = sign

    ent(0, 0, const=True); ent(0, 1)
    term(0, 0, 0, -1); term(0, 0, 1, +1)
    ent(1, 2, const=True); ent(2, 3, const=True)
    for i in range(n):
        ent(3 + 3 * i, 4 + 3 * i)
        term(1 + i, 3 + 3 * i, 4 + 3 * i, +1)
        ent(4 + 3 * i, 5 + 3 * i, const=True)
        ent(5 + 3 * i, 6 + 3 * i, const=True)
    off = 8 + 3 * n
    for i in range(n + 1):
        ent(3 + 3 * i, off + 3 * i)
        term(101 + i, 3 + 3 * i, off + 3 * i, +1)
        ent(off + 3 * i, off + 1 + 3 * i, const=True)
        ent(off + 1 + 3 * i, off + 2 + 3 * i, const=True)
        ent(off + 2 + 3 * i, 4 + 3 * i)
        ent(off + 2 + 3 * i, off + 3 * i, const=True)
        term(203 + i, off + 2 + 3 * i, 4 + 3 * i, +1)
        term(203 + i, off + 2 + 3 * i, off + 3 * i, -1)
    ent(303, 304)
    term(202, 303, 304, +1)
    for i in range(n):
        for j in range(i + 1, n + 1):
            r, c = 3 + 3 * i, 4 + 3 * j
            flags[r, c] |= _F_MASK | _F_CONST | _F_DEL
            ke[r, c] = 1 + (j - i)
    t1 = 8 + 3 * n + 3 * (n + 1)
    for r, c in ((304, 305), (305, 306), (306, 307), (307, 307), (307, t1), (t1, t1)):
        ent(r, c, const=True)
    return flags, ke, rowone, colsign


_TABLES = _static_tables()


def _cell_body(inp_ref, alpha_ref, count_ref, loglik_ref, w_ref, ek_ref, ik_ref,
               flags_ref, ke_ref, rowone_ref, colsign_ref,
               alpha_out_ref, count_out_ref, loglik_out_ref):
    w = w_ref[...]

    right = w[:_NTERM, :] * colsign_ref[...].astype(jnp.float32)
    Vvar = jnp.dot(rowone_ref[...], right,
                   preferred_element_type=jnp.float32, precision=_HIGH)

    w304 = w_ref[304, 0]
    loga = jnp.log(jnp.abs(w304))
    sgn = jnp.sign(w304)
    ke = ke_ref[...].astype(jnp.float32)
    odd = ke - 2.0 * jnp.floor(ke * 0.5)
    pw = jnp.exp(ke * loga) * (odd * sgn + (1.0 - odd))

    flags = flags_ref[...].astype(jnp.int32)
    constm = (flags & _F_CONST).astype(jnp.float32)
    maskm = ((flags >> 1) & 1).astype(jnp.float32)
    delm = ((flags >> 2) & 1).astype(jnp.float32)

    V = Vvar + constm - delm * pw

    gmax = jnp.max(jnp.where(maskm > 0.0, V, -jnp.inf))
    E = maskm * jnp.exp(V - gmax)
    rowsum = jnp.sum(E, axis=1, keepdims=True)
    A = E * (1.0 / rowsum)

    x = ek_ref[...][:, :_NEMIT]
    xm = jnp.max(x, axis=1, keepdims=True)
    Bexp = jnp.exp(x - xm)
    B = Bexp * (1.0 / jnp.sum(Bexp, axis=1, keepdims=True))
    emis = jax.lax.dot_general(inp_ref[...], B, (((1,), (1,)), ((), ())),
                               preferred_element_type=jnp.float32, precision=_HIGH)

    ik = ik_ref[...]
    ikm = jnp.max(ik)
    pexp = jnp.exp(ik - ikm)
    pi = pexp * (1.0 / jnp.sum(pexp))

    alphaA = jnp.dot(alpha_ref[...], A, preferred_element_type=jnp.float32, precision=_HIGH)
    count = count_ref[...]
    first = count == 0.0
    alpha_new = jnp.where(first, pi, alphaA) * emis
    Z = jnp.sum(alpha_new, axis=1, keepdims=True) + 1e-30
    alpha_out_ref[...] = alpha_new / Z
    count_out_ref[...] = count + 1.0
    loglik_out_ref[...] = loglik_ref[...] + jnp.log(Z)


def kernel(inputs, alpha, count, loglik, transition_kernel, emission_kernel, init_kernel):
    batch = inputs.shape[0]
    w = transition_kernel.reshape(_NTRANS, 1)
    ek = emission_kernel.reshape(_S, _EMITC)
    ik = init_kernel.reshape(1, _S)
    out = pl.pallas_call(
        _cell_body,
        out_shape=(
            jax.ShapeDtypeStruct((batch, _S), jnp.float32),
            jax.ShapeDtypeStruct((batch, 1), jnp.float32),
            jax.ShapeDtypeStruct((batch, 1), jnp.float32),
        ),
    )(inputs, alpha, count, loglik, w, ek, ik,
      *(jnp.asarray(t) for t in _TABLES))
    return out


# speedup vs baseline: 1.5424x; 1.3825x over previous
import numpy as np
import jax
import jax.numpy as jnp
from jax.experimental import pallas as pl
from jax.experimental.pallas import tpu as pltpu

_N = 100
_S = 6 * _N + 12
_NTRANS = 3 * _N + 5
_NTERM = 304
_NEMIT = 126
_EMITC = 6 ** 3

_NEG = -1e30


def _static_tables():
    n, S = _N, _S
    base = np.full((S, S), _NEG, np.float32)
    ke = np.ones((S, S), np.int8)
    rowone = np.zeros((S, _NTERM), np.float32)
    colsign = np.zeros((_NTERM, S), np.float32)

    def ent(r, c, const=0.0):
        base[r, c] = const

    def term(t, r, c, sign):
        rowone[r, t] = 1.0
        colsign[t, c] = sign

    ent(0, 0, 1.0); ent(0, 1)
    term(0, 0, 0, -1); term(0, 0, 1, +1)
    ent(1, 2, 1.0); ent(2, 3, 1.0)
    for i in range(n):
        ent(3 + 3 * i, 4 + 3 * i)
        term(1 + i, 3 + 3 * i, 4 + 3 * i, +1)
        ent(4 + 3 * i, 5 + 3 * i, 1.0)
        ent(5 + 3 * i, 6 + 3 * i, 1.0)
    off = 8 + 3 * n
    for i in range(n + 1):
        ent(3 + 3 * i, off + 3 * i)
        term(101 + i, 3 + 3 * i, off + 3 * i, +1)
        ent(off + 3 * i, off + 1 + 3 * i, 1.0)
        ent(off + 1 + 3 * i, off + 2 + 3 * i, 1.0)
        ent(off + 2 + 3 * i, 4 + 3 * i)
        ent(off + 2 + 3 * i, off + 3 * i, 1.0)
        term(203 + i, off + 2 + 3 * i, 4 + 3 * i, +1)
        term(203 + i, off + 2 + 3 * i, off + 3 * i, -1)
    ent(303, 304)
    term(202, 303, 304, +1)
    for i in range(n):
        for j in range(i + 1, n + 1):
            r, c = 3 + 3 * i, 4 + 3 * j
            base[r, c] = 1.0
            ke[r, c] = 1 + (j - i)
    t1 = 8 + 3 * n + 3 * (n + 1)
    for r, c in ((304, 305), (305, 306), (306, 307), (307, 307), (307, t1), (t1, t1)):
        ent(r, c, 1.0)
    return base, ke, rowone, colsign


_TABLES = _static_tables()


def _cell_body(inp_ref, alpha_ref, count_ref, loglik_ref, w_ref, ek_ref, ik_ref,
               base_ref, ke_ref, rowone_ref, colsign_ref,
               alpha_out_ref, count_out_ref, loglik_out_ref):
    w = w_ref[...]

    right = (w[:_NTERM, :] * colsign_ref[...]).astype(jnp.bfloat16)
    Vvar = jnp.dot(rowone_ref[...], right, preferred_element_type=jnp.float32)

    w304 = w_ref[304, 0]
    loga = jnp.log(jnp.abs(w304))
    sgn = jnp.sign(w304)
    ke = ke_ref[...].astype(jnp.float32)
    odd = ke - 2.0 * jnp.floor(ke * 0.5)
    pw = jnp.exp(ke * loga) * (odd * sgn + (1.0 - odd))
    delm = (ke > 1.5).astype(jnp.float32)

    V = base_ref[...] + Vvar - delm * pw

    gmax = jnp.max(V)
    E = jnp.exp(V - gmax)
    rowsum = jnp.sum(E, axis=1, keepdims=True)
    A = (E * (1.0 / rowsum)).astype(jnp.bfloat16)

    x = ek_ref[...][:, :_NEMIT]
    xm = jnp.max(x, axis=1, keepdims=True)
    Bexp = jnp.exp(x - xm)
    B = (Bexp * (1.0 / jnp.sum(Bexp, axis=1, keepdims=True))).astype(jnp.bfloat16)
    emis = jax.lax.dot_general(inp_ref[...], B, (((1,), (1,)), ((), ())),
                               preferred_element_type=jnp.float32)

    ik = ik_ref[...]
    ikm = jnp.max(ik)
    pexp = jnp.exp(ik - ikm)
    pi = pexp * (1.0 / jnp.sum(pexp))

    alphaA = jnp.dot(alpha_ref[...], A, preferred_element_type=jnp.float32)
    count = count_ref[...]
    first = count == 0.0
    alpha_new = jnp.where(first, pi, alphaA) * emis
    Z = jnp.sum(alpha_new, axis=1, keepdims=True) + 1e-30
    alpha_out_ref[...] = alpha_new / Z
    count_out_ref[...] = count + 1.0
    loglik_out_ref[...] = loglik_ref[...] + jnp.log(Z)


def kernel(inputs, alpha, count, loglik, transition_kernel, emission_kernel, init_kernel):
    batch = inputs.shape[0]
    w = transition_kernel.reshape(_NTRANS, 1)
    ek = emission_kernel.reshape(_S, _EMITC)
    ik = init_kernel.reshape(1, _S)
    out = pl.pallas_call(
        _cell_body,
        out_shape=(
            jax.ShapeDtypeStruct((batch, _S), jnp.float32),
            jax.ShapeDtypeStruct((batch, 1), jnp.float32),
            jax.ShapeDtypeStruct((batch, 1), jnp.float32),
        ),
    )(inputs.astype(jnp.bfloat16), alpha.astype(jnp.bfloat16), count, loglik,
      w, ek, ik,
      jnp.asarray(_TABLES[0]), jnp.asarray(_TABLES[1]),
      jnp.asarray(_TABLES[2], jnp.bfloat16), jnp.asarray(_TABLES[3]))
    return out
